# baseline (speedup 1.0000x reference)
import math

import jax
import jax.numpy as jnp
from jax import lax
from jax.experimental import pallas as pl

N = 4096
SIZE = 32000
SMOOTHING = 0.1
CONFIDENCE = 1.0 - SMOOTHING
FILL = SMOOTHING / (SIZE - 1)
K_CONST = N * (SIZE - 1) * FILL * math.log(FILL) + N * CONFIDENCE * math.log(CONFIDENCE)

NSTREAM = 16
SLAB = 8
STEP_ROWS = NSTREAM * SLAB


def _main_body(*refs):
    x_refs, tgt_ref, s_ref, g_ref = refs[:NSTREAM], refs[NSTREAM], refs[NSTREAM + 1], refs[NSTREAM + 2]
    i = pl.program_id(0)
    t_all = tgt_ref[...]
    col = lax.broadcasted_iota(jnp.int32, (SLAB, SIZE), 1)

    s = jnp.float32(0.0)
    for g in range(NSTREAM // 4):
        p = (
            x_refs[4 * g][...]
            * x_refs[4 * g + 1][...]
            * x_refs[4 * g + 2][...]
            * x_refs[4 * g + 3][...]
        )
        s += jnp.sum(jnp.log(p))

    for j in range(NSTREAM):
        xb = x_refs[j][...]
        t = t_all[j * SLAB : (j + 1) * SLAB, :]
        picked = jnp.where(col == t, xb, 0.0)
        g_ref[j * SLAB : (j + 1) * SLAB, :] = jnp.sum(picked, axis=1, keepdims=True)

    @pl.when(i == 0)
    def _():
        s_ref[...] = jnp.zeros_like(s_ref)

    s_ref[...] += s


def _combine_body(g_ref, s_ref, o_ref):
    s_tgt = jnp.sum(jnp.log(g_ref[...]))
    o_ref[...] = K_CONST - FILL * s_ref[...] - (CONFIDENCE - FILL) * s_tgt


def kernel(x, target):
    s_all, g = pl.pallas_call(
        _main_body,
        grid=(N // STEP_ROWS,),
        in_specs=[
            pl.BlockSpec((SLAB, SIZE), (lambda i, j=j: (i * NSTREAM + j, 0)))
            for j in range(NSTREAM)
        ]
        + [pl.BlockSpec((STEP_ROWS, 1), lambda i: (i, 0))],
        out_specs=[
            pl.BlockSpec((1, 1), lambda i: (0, 0)),
            pl.BlockSpec((STEP_ROWS, 1), lambda i: (i, 0)),
        ],
        out_shape=[
            jax.ShapeDtypeStruct((1, 1), jnp.float32),
            jax.ShapeDtypeStruct((N, 1), jnp.float32),
        ],
    )(*([x] * NSTREAM), target.reshape(N, 1))

    loss = pl.pallas_call(
        _combine_body,
        in_specs=[
            pl.BlockSpec((32, N // 32), lambda: (0, 0)),
            pl.BlockSpec((1, 1), lambda: (0, 0)),
        ],
        out_specs=pl.BlockSpec((1, 1), lambda: (0, 0)),
        out_shape=jax.ShapeDtypeStruct((1, 1), jnp.float32),
    )(g.reshape(32, N // 32), s_all)

    return loss.reshape(())
